# baseline (speedup 1.0000x reference)
import functools

import jax
import jax.numpy as jnp
from jax.experimental import pallas as pl

_BETA = 0.25
_F32 = jnp.float32
_KC = 512


def _mm(a, b):
    return jax.lax.dot_general(a, b, (((1,), (0,)), ((), ())),
                               preferred_element_type=_F32)


def _mm_t(a, b):
    return jax.lax.dot_general(a, b, (((1,), (1,)), ((), ())),
                               preferred_element_type=_F32)


def _rqvae_tile(x_ref, w1_ref, b1_ref, w2_ref, b2_ref, w3_ref, b3_ref,
                v1_ref, c1_ref, v2_ref, c2_ref, v3_ref, c3_ref,
                cbs_ref, out_ref, *, num_levels, k):
    xt = x_ref[...]
    h = jnp.maximum(_mm(xt, w1_ref[...]) + b1_ref[...], 0.0)
    h = jnp.maximum(_mm(h, w2_ref[...]) + b2_ref[...], 0.0)
    res = _mm(h, w3_ref[...]) + b3_ref[...]

    bt, dim = res.shape
    nchunks = k // _KC
    iota_c = jax.lax.broadcasted_iota(jnp.int32, (bt, _KC), 1)
    rq = jnp.zeros((bt, 1), dtype=_F32)
    z_hat = jnp.zeros_like(res)
    for l in range(num_levels):
        aug_res = jnp.concatenate([res, jnp.ones((bt, 1), _F32)], axis=1)

        def _scan_chunk(i, carry, l=l, aug_res=aug_res):
            best, bestid = carry
            cbc = cbs_ref[l, pl.ds(i * _KC, _KC), :]
            aug_cbc = jnp.concatenate(
                [cbc * -2.0, jnp.sum(cbc * cbc, axis=1, keepdims=True)], axis=1)
            score = _mm_t(aug_res, aug_cbc)
            m = jnp.min(score, axis=1, keepdims=True)
            a = jnp.argmin(score, axis=1).reshape(bt, 1).astype(jnp.int32)
            upd = m < best
            return (jnp.where(upd, m, best),
                    jnp.where(upd, a + i * _KC, bestid))

        best0 = jnp.full((bt, 1), jnp.inf, _F32)
        id0 = jnp.zeros((bt, 1), jnp.int32)
        _, bestid = jax.lax.fori_loop(0, nchunks, _scan_chunk, (best0, id0))

        def _gather_chunk(i, cw, l=l, bestid=bestid):
            cbc = cbs_ref[l, pl.ds(i * _KC, _KC), :]
            onehot = (iota_c + i * _KC == bestid).astype(_F32)
            return cw + _mm(onehot, cbc)

        cw = jax.lax.fori_loop(0, nchunks, _gather_chunk,
                               jnp.zeros((bt, dim), _F32))
        diff = cw - res
        rq = rq + jnp.sum(diff * diff, axis=1, keepdims=True)
        z_hat = z_hat + cw
        res = -diff

    rq = rq * (1.0 + _BETA)
    d = jnp.maximum(_mm(z_hat, v1_ref[...]) + c1_ref[...], 0.0)
    d = jnp.maximum(_mm(d, v2_ref[...]) + c2_ref[...], 0.0)
    x_hat = _mm(d, v3_ref[...]) + c3_ref[...]
    e = x_hat - xt
    recon = jnp.sum(e * e, axis=1, keepdims=True)
    out_ref[...] = jnp.full((1, 1, 128), jnp.sum(recon + rq), dtype=_F32)


@jax.jit
def kernel(x, temperature, enc_params, dec_params, codebooks):
    del temperature
    batch, in_dim = x.shape
    cbs = jnp.stack(codebooks, axis=0)
    num_levels, k, _ = cbs.shape

    (w1, b1), (w2, b2), (w3, b3) = enc_params
    (v1, c1), (v2, c2), (v3, c3) = dec_params
    row = lambda b: b.reshape(1, -1)

    bt = 256
    grid = batch // bt
    full = lambda a: pl.BlockSpec(a.shape, lambda i: (0,) * a.ndim)

    partials = pl.pallas_call(
        functools.partial(_rqvae_tile, num_levels=num_levels, k=k),
        grid=(grid,),
        in_specs=[
            pl.BlockSpec((bt, in_dim), lambda i: (i, 0)),
            full(w1), full(row(b1)), full(w2), full(row(b2)),
            full(w3), full(row(b3)),
            full(v1), full(row(c1)), full(v2), full(row(c2)),
            full(v3), full(row(c3)),
            full(cbs),
        ],
        out_specs=pl.BlockSpec((1, 1, 128), lambda i: (i, 0, 0)),
        out_shape=jax.ShapeDtypeStruct((grid, 1, 128), _F32),
    )(x, w1, row(b1), w2, row(b2), w3, row(b3),
      v1, row(c1), v2, row(c2), v3, row(c3), cbs)

    return jnp.sum(partials[:, 0, 0]) / batch


# speedup vs baseline: 1.1675x; 1.1675x over previous
import functools

import jax
import jax.numpy as jnp
from jax.experimental import pallas as pl

_BETA = 0.25
_F32 = jnp.float32
_KC = 512


def _mm(a, b):
    return jax.lax.dot_general(a, b, (((1,), (0,)), ((), ())),
                               preferred_element_type=_F32)


def _mm_t(a, b):
    return jax.lax.dot_general(a, b, (((1,), (1,)), ((), ())),
                               preferred_element_type=_F32)


def _augment_level(cb_ref, out_ref):
    cb = cb_ref[0]
    c2 = jnp.sum(cb * cb, axis=1, keepdims=True)
    out_ref[0] = jnp.concatenate([cb * -2.0, c2], axis=1)


def _rqvae_tile(x_ref, w1_ref, b1_ref, w2_ref, b2_ref, w3_ref, b3_ref,
                v1_ref, c1_ref, v2_ref, c2_ref, v3_ref, c3_ref,
                aug_ref, out_ref, *, num_levels, k):
    xt = x_ref[...]
    h = jnp.maximum(_mm(xt, w1_ref[...]) + b1_ref[...], 0.0)
    h = jnp.maximum(_mm(h, w2_ref[...]) + b2_ref[...], 0.0)
    res = _mm(h, w3_ref[...]) + b3_ref[...]

    bt, dim = res.shape
    nchunks = k // _KC
    rq = jnp.zeros((bt, 1), dtype=_F32)
    z_hat = jnp.zeros_like(res)
    for l in range(num_levels):
        aug_res = jnp.concatenate([res, jnp.ones((bt, 1), _F32)], axis=1)

        def _scan_chunk(i, carry, l=l, aug_res=aug_res):
            best, bestc = carry
            augc = aug_ref[l, pl.ds(i * _KC, _KC), :]
            score = _mm_t(aug_res, augc)
            m = jnp.min(score, axis=1, keepdims=True)
            upd = m < best
            return jnp.where(upd, m, best), jnp.where(upd, i, bestc)

        best0 = jnp.full((bt, 1), jnp.inf, _F32)
        c0 = jnp.zeros((bt, 1), jnp.int32)
        best, bestc = jax.lax.fori_loop(0, nchunks, _scan_chunk, (best0, c0))

        def _gather_chunk(i, acc, l=l, aug_res=aug_res, best=best, bestc=bestc):
            augc = aug_ref[l, pl.ds(i * _KC, _KC), :]
            score = _mm_t(aug_res, augc)
            oh = ((score == best) & (bestc == i)).astype(_F32)
            return acc + _mm(oh, augc[:, :dim])

        acc = jax.lax.fori_loop(0, nchunks, _gather_chunk,
                                jnp.zeros((bt, dim), _F32))
        cw = acc * -0.5
        diff = cw - res
        rq = rq + jnp.sum(diff * diff, axis=1, keepdims=True)
        z_hat = z_hat + cw
        res = -diff

    rq = rq * (1.0 + _BETA)
    d = jnp.maximum(_mm(z_hat, v1_ref[...]) + c1_ref[...], 0.0)
    d = jnp.maximum(_mm(d, v2_ref[...]) + c2_ref[...], 0.0)
    x_hat = _mm(d, v3_ref[...]) + c3_ref[...]
    e = x_hat - xt
    recon = jnp.sum(e * e, axis=1, keepdims=True)
    out_ref[...] = jnp.full((1, 1, 128), jnp.sum(recon + rq), dtype=_F32)


@jax.jit
def kernel(x, temperature, enc_params, dec_params, codebooks):
    del temperature
    batch, in_dim = x.shape
    cbs = jnp.stack(codebooks, axis=0)
    num_levels, k, dim = cbs.shape

    aug = pl.pallas_call(
        _augment_level,
        grid=(num_levels,),
        in_specs=[pl.BlockSpec((1, k, dim), lambda l: (l, 0, 0))],
        out_specs=pl.BlockSpec((1, k, dim + 1), lambda l: (l, 0, 0)),
        out_shape=jax.ShapeDtypeStruct((num_levels, k, dim + 1), _F32),
    )(cbs)

    (w1, b1), (w2, b2), (w3, b3) = enc_params
    (v1, c1), (v2, c2), (v3, c3) = dec_params
    row = lambda b: b.reshape(1, -1)

    bt = 256
    grid = batch // bt
    full = lambda a: pl.BlockSpec(a.shape, lambda i: (0,) * a.ndim)

    partials = pl.pallas_call(
        functools.partial(_rqvae_tile, num_levels=num_levels, k=k),
        grid=(grid,),
        in_specs=[
            pl.BlockSpec((bt, in_dim), lambda i: (i, 0)),
            full(w1), full(row(b1)), full(w2), full(row(b2)),
            full(w3), full(row(b3)),
            full(v1), full(row(c1)), full(v2), full(row(c2)),
            full(v3), full(row(c3)),
            full(aug),
        ],
        out_specs=pl.BlockSpec((1, 1, 128), lambda i: (i, 0, 0)),
        out_shape=jax.ShapeDtypeStruct((grid, 1, 128), _F32),
    )(x, w1, row(b1), w2, row(b2), w3, row(b3),
      v1, row(c1), v2, row(c2), v3, row(c3), aug)

    return jnp.sum(partials[:, 0, 0]) / batch


# speedup vs baseline: 1.7708x; 1.5167x over previous
import functools

import jax
import jax.numpy as jnp
from jax.experimental import pallas as pl

_BETA = 0.25
_F32 = jnp.float32
_BF16 = jnp.bfloat16
_KC = 1024


def _mm(a, b):
    return jax.lax.dot_general(a, b, (((1,), (0,)), ((), ())),
                               preferred_element_type=_F32)


def _mm_t(a, b):
    return jax.lax.dot_general(a, b, (((1,), (1,)), ((), ())),
                               preferred_element_type=_F32)


def _augment_level(cb_ref, out_ref):
    cb = cb_ref[0]
    c2 = jnp.sum(cb * cb, axis=1, keepdims=True)
    c2_hi = c2.astype(_BF16).astype(_F32)
    out_ref[0] = jnp.concatenate(
        [cb * -2.0, c2_hi, c2 - c2_hi], axis=1).astype(_BF16)


def _rqvae_tile(x_ref, w1_ref, b1_ref, w2_ref, b2_ref, w3_ref, b3_ref,
                v1_ref, c1_ref, v2_ref, c2_ref, v3_ref, c3_ref,
                aug_ref, cbs_ref, out_ref, *, num_levels, k):
    xt = x_ref[...]
    h = jnp.maximum(_mm(xt, w1_ref[...]) + b1_ref[...], 0.0)
    h = jnp.maximum(_mm(h, w2_ref[...]) + b2_ref[...], 0.0)
    res = _mm(h, w3_ref[...]) + b3_ref[...]

    bt, dim = res.shape
    nchunks = k // _KC
    rq = jnp.zeros((bt, 1), dtype=_F32)
    z_hat = jnp.zeros_like(res)
    for l in range(num_levels):
        aug_res = jnp.concatenate(
            [res, jnp.ones((bt, 2), _F32)], axis=1).astype(_BF16)

        def _scan_chunk(i, carry, l=l, aug_res=aug_res):
            best, bestc = carry
            augc = aug_ref[l, pl.ds(i * _KC, _KC), :]
            score = _mm_t(aug_res, augc)
            m = jnp.min(score, axis=1, keepdims=True)
            upd = m < best
            return jnp.where(upd, m, best), jnp.where(upd, i, bestc)

        best0 = jnp.full((bt, 1), jnp.inf, _F32)
        c0 = jnp.zeros((bt, 1), jnp.int32)
        best, bestc = jax.lax.fori_loop(0, nchunks, _scan_chunk, (best0, c0))

        def _gather_chunk(i, acc, l=l, aug_res=aug_res, best=best, bestc=bestc):
            augc = aug_ref[l, pl.ds(i * _KC, _KC), :]
            score = _mm_t(aug_res, augc)
            oh = ((score == best) & (bestc == i)).astype(_F32)
            return acc + _mm(oh, cbs_ref[l, pl.ds(i * _KC, _KC), :])

        cw = jax.lax.fori_loop(0, nchunks, _gather_chunk,
                               jnp.zeros((bt, dim), _F32))
        diff = cw - res
        rq = rq + jnp.sum(diff * diff, axis=1, keepdims=True)
        z_hat = z_hat + cw
        res = -diff

    rq = rq * (1.0 + _BETA)
    d = jnp.maximum(_mm(z_hat, v1_ref[...]) + c1_ref[...], 0.0)
    d = jnp.maximum(_mm(d, v2_ref[...]) + c2_ref[...], 0.0)
    x_hat = _mm(d, v3_ref[...]) + c3_ref[...]
    e = x_hat - xt
    recon = jnp.sum(e * e, axis=1, keepdims=True)
    out_ref[...] = jnp.full((1, 1, 128), jnp.sum(recon + rq), dtype=_F32)


@jax.jit
def kernel(x, temperature, enc_params, dec_params, codebooks):
    del temperature
    batch, in_dim = x.shape
    cbs = jnp.stack(codebooks, axis=0)
    num_levels, k, dim = cbs.shape

    aug = pl.pallas_call(
        _augment_level,
        grid=(num_levels,),
        in_specs=[pl.BlockSpec((1, k, dim), lambda l: (l, 0, 0))],
        out_specs=pl.BlockSpec((1, k, dim + 2), lambda l: (l, 0, 0)),
        out_shape=jax.ShapeDtypeStruct((num_levels, k, dim + 2), _BF16),
    )(cbs)

    (w1, b1), (w2, b2), (w3, b3) = enc_params
    (v1, c1), (v2, c2), (v3, c3) = dec_params
    row = lambda b: b.reshape(1, -1)

    bt = 256
    grid = batch // bt
    full = lambda a: pl.BlockSpec(a.shape, lambda i: (0,) * a.ndim)

    partials = pl.pallas_call(
        functools.partial(_rqvae_tile, num_levels=num_levels, k=k),
        grid=(grid,),
        in_specs=[
            pl.BlockSpec((bt, in_dim), lambda i: (i, 0)),
            full(w1), full(row(b1)), full(w2), full(row(b2)),
            full(w3), full(row(b3)),
            full(v1), full(row(c1)), full(v2), full(row(c2)),
            full(v3), full(row(c3)),
            full(aug), full(cbs),
        ],
        out_specs=pl.BlockSpec((1, 1, 128), lambda i: (i, 0, 0)),
        out_shape=jax.ShapeDtypeStruct((grid, 1, 128), _F32),
    )(x, w1, row(b1), w2, row(b2), w3, row(b3),
      v1, row(c1), v2, row(c2), v3, row(c3), aug, cbs)

    return jnp.sum(partials[:, 0, 0]) / batch


# speedup vs baseline: 2.2506x; 1.2710x over previous
import functools

import jax
import jax.numpy as jnp
from jax.experimental import pallas as pl

_BETA = 0.25
_F32 = jnp.float32
_BF16 = jnp.bfloat16
_KC = 1024


def _mm(a, b):
    return jax.lax.dot_general(a, b, (((1,), (0,)), ((), ())),
                               preferred_element_type=_F32)


def _mm_t(a, b):
    return jax.lax.dot_general(a, b, (((1,), (1,)), ((), ())),
                               preferred_element_type=_F32)


def _augment_level(cb_ref, aug_ref, cbhl_ref):
    cb = cb_ref[0]
    c2 = jnp.sum(cb * cb, axis=1, keepdims=True)
    c2_hi = c2.astype(_BF16).astype(_F32)
    aug_ref[0] = jnp.concatenate(
        [cb * -2.0, c2_hi, c2 - c2_hi], axis=1).astype(_BF16)
    hi = cb.astype(_BF16)
    cbhl_ref[0] = jnp.concatenate(
        [hi, (cb - hi.astype(_F32)).astype(_BF16)], axis=1)


def _rqvae_tile(x_ref, w1_ref, b1_ref, w2_ref, b2_ref, w3_ref, b3_ref,
                v1_ref, c1_ref, v2_ref, c2_ref, v3_ref, c3_ref,
                aug_ref, cbhl_ref, out_ref, *, num_levels, k):
    xt = x_ref[...]
    h = jnp.maximum(_mm(xt, w1_ref[...]) + b1_ref[...], 0.0)
    h = jnp.maximum(_mm(h, w2_ref[...]) + b2_ref[...], 0.0)
    res = _mm(h, w3_ref[...]) + b3_ref[...]

    bt, dim = res.shape
    nchunks = k // _KC
    rq = jnp.zeros((bt, 1), dtype=_F32)
    z_hat = jnp.zeros_like(res)
    for l in range(num_levels):
        aug_res = jnp.concatenate(
            [res, jnp.ones((bt, 2), _F32)], axis=1).astype(_BF16)

        def _scan_chunk(i, carry, l=l, aug_res=aug_res):
            best, bestc = carry
            augc = aug_ref[l, pl.ds(i * _KC, _KC), :]
            score = _mm_t(aug_res, augc)
            m = jnp.min(score, axis=1, keepdims=True)
            upd = m < best
            return jnp.where(upd, m, best), jnp.where(upd, i, bestc)

        best0 = jnp.full((bt, 1), jnp.inf, _F32)
        c0 = jnp.zeros((bt, 1), jnp.int32)
        best, bestc = jax.lax.fori_loop(0, nchunks, _scan_chunk, (best0, c0))

        def _gather_chunk(i, acc, l=l, aug_res=aug_res, best=best, bestc=bestc):
            augc = aug_ref[l, pl.ds(i * _KC, _KC), :]
            score = _mm_t(aug_res, augc)
            oh = ((score == best) & (bestc == i)).astype(_BF16)
            return acc + _mm(oh, cbhl_ref[l, pl.ds(i * _KC, _KC), :])

        hl = jax.lax.fori_loop(0, nchunks, _gather_chunk,
                               jnp.zeros((bt, 2 * dim), _F32))
        cw = hl[:, :dim] + hl[:, dim:]
        diff = cw - res
        rq = rq + jnp.sum(diff * diff, axis=1, keepdims=True)
        z_hat = z_hat + cw
        res = -diff

    rq = rq * (1.0 + _BETA)
    d = jnp.maximum(_mm(z_hat, v1_ref[...]) + c1_ref[...], 0.0)
    d = jnp.maximum(_mm(d, v2_ref[...]) + c2_ref[...], 0.0)
    x_hat = _mm(d, v3_ref[...]) + c3_ref[...]
    e = x_hat - xt
    recon = jnp.sum(e * e, axis=1, keepdims=True)
    out_ref[...] = jnp.full((1, 1, 128), jnp.sum(recon + rq), dtype=_F32)


@jax.jit
def kernel(x, temperature, enc_params, dec_params, codebooks):
    del temperature
    batch, in_dim = x.shape
    cbs = jnp.stack(codebooks, axis=0)
    num_levels, k, dim = cbs.shape

    aug, cbhl = pl.pallas_call(
        _augment_level,
        grid=(num_levels,),
        in_specs=[pl.BlockSpec((1, k, dim), lambda l: (l, 0, 0))],
        out_specs=[pl.BlockSpec((1, k, dim + 2), lambda l: (l, 0, 0)),
                   pl.BlockSpec((1, k, 2 * dim), lambda l: (l, 0, 0))],
        out_shape=[jax.ShapeDtypeStruct((num_levels, k, dim + 2), _BF16),
                   jax.ShapeDtypeStruct((num_levels, k, 2 * dim), _BF16)],
    )(cbs)

    (w1, b1), (w2, b2), (w3, b3) = enc_params
    (v1, c1), (v2, c2), (v3, c3) = dec_params
    row = lambda b: b.reshape(1, -1)

    bt = 512
    grid = batch // bt
    full = lambda a: pl.BlockSpec(a.shape, lambda i: (0,) * a.ndim)

    partials = pl.pallas_call(
        functools.partial(_rqvae_tile, num_levels=num_levels, k=k),
        grid=(grid,),
        in_specs=[
            pl.BlockSpec((bt, in_dim), lambda i: (i, 0)),
            full(w1), full(row(b1)), full(w2), full(row(b2)),
            full(w3), full(row(b3)),
            full(v1), full(row(c1)), full(v2), full(row(c2)),
            full(v3), full(row(c3)),
            full(aug), full(cbhl),
        ],
        out_specs=pl.BlockSpec((1, 1, 128), lambda i: (i, 0, 0)),
        out_shape=jax.ShapeDtypeStruct((grid, 1, 128), _F32),
    )(x, w1, row(b1), w2, row(b2), w3, row(b3),
      v1, row(c1), v2, row(c2), v3, row(c3), aug, cbhl)

    return jnp.sum(partials[:, 0, 0]) / batch


# speedup vs baseline: 2.3122x; 1.0273x over previous
import functools

import jax
import jax.numpy as jnp
from jax.experimental import pallas as pl

_BETA = 0.25
_F32 = jnp.float32
_BF16 = jnp.bfloat16
_KC = 1024


def _mm(a, b):
    return jax.lax.dot_general(a, b, (((1,), (0,)), ((), ())),
                               preferred_element_type=_F32)


def _mm_t(a, b):
    return jax.lax.dot_general(a, b, (((1,), (1,)), ((), ())),
                               preferred_element_type=_F32)


def _augment_level(cb_ref, aug_ref, cbhl_ref):
    cb = cb_ref[0]
    c2 = jnp.sum(cb * cb, axis=1, keepdims=True)
    c2_hi = c2.astype(_BF16).astype(_F32)
    aug_ref[0] = jnp.concatenate(
        [cb * -2.0, c2_hi, c2 - c2_hi], axis=1).astype(_BF16)
    hi = cb.astype(_BF16)
    cbhl_ref[0] = jnp.concatenate(
        [hi, (cb - hi.astype(_F32)).astype(_BF16)], axis=1)


def _rqvae_tile(x_ref, w1_ref, b1_ref, w2_ref, b2_ref, w3_ref, b3_ref,
                v1_ref, c1_ref, v2_ref, c2_ref, v3_ref, c3_ref,
                aug_ref, cbhl_ref, out_ref, *, num_levels, k):
    xt = x_ref[...]
    h = jnp.maximum(_mm(xt, w1_ref[...]) + b1_ref[...], 0.0)
    h = jnp.maximum(_mm(h, w2_ref[...]) + b2_ref[...], 0.0)
    res = _mm(h, w3_ref[...]) + b3_ref[...]

    bt, dim = res.shape
    nchunks = k // _KC
    rq = jnp.zeros((bt, 1), dtype=_F32)
    z_hat = jnp.zeros_like(res)
    for l in range(num_levels):
        aug_res = jnp.concatenate(
            [res, jnp.ones((bt, 2), _F32)], axis=1).astype(_BF16)

        def _chunk(i, carry, l=l, aug_res=aug_res):
            best, hl = carry
            augc = aug_ref[l, pl.ds(i * _KC, _KC), :]
            score = _mm_t(aug_res, augc)
            m = jnp.min(score, axis=1, keepdims=True)
            oh = (score == m).astype(_BF16)
            hl_i = _mm(oh, cbhl_ref[l, pl.ds(i * _KC, _KC), :])
            upd = m < best
            return (jnp.where(upd, m, best), jnp.where(upd, hl_i, hl))

        best0 = jnp.full((bt, 1), jnp.inf, _F32)
        hl0 = jnp.zeros((bt, 2 * dim), _F32)
        _, hl = jax.lax.fori_loop(0, nchunks, _chunk, (best0, hl0))
        cw = hl[:, :dim] + hl[:, dim:]
        diff = cw - res
        rq = rq + jnp.sum(diff * diff, axis=1, keepdims=True)
        z_hat = z_hat + cw
        res = -diff

    rq = rq * (1.0 + _BETA)
    d = jnp.maximum(_mm(z_hat, v1_ref[...]) + c1_ref[...], 0.0)
    d = jnp.maximum(_mm(d, v2_ref[...]) + c2_ref[...], 0.0)
    x_hat = _mm(d, v3_ref[...]) + c3_ref[...]
    e = x_hat - xt
    recon = jnp.sum(e * e, axis=1, keepdims=True)
    out_ref[...] = jnp.full((1, 1, 128), jnp.sum(recon + rq), dtype=_F32)


@jax.jit
def kernel(x, temperature, enc_params, dec_params, codebooks):
    del temperature
    batch, in_dim = x.shape
    cbs = jnp.stack(codebooks, axis=0)
    num_levels, k, dim = cbs.shape

    aug, cbhl = pl.pallas_call(
        _augment_level,
        grid=(num_levels,),
        in_specs=[pl.BlockSpec((1, k, dim), lambda l: (l, 0, 0))],
        out_specs=[pl.BlockSpec((1, k, dim + 2), lambda l: (l, 0, 0)),
                   pl.BlockSpec((1, k, 2 * dim), lambda l: (l, 0, 0))],
        out_shape=[jax.ShapeDtypeStruct((num_levels, k, dim + 2), _BF16),
                   jax.ShapeDtypeStruct((num_levels, k, 2 * dim), _BF16)],
    )(cbs)

    (w1, b1), (w2, b2), (w3, b3) = enc_params
    (v1, c1), (v2, c2), (v3, c3) = dec_params
    row = lambda b: b.reshape(1, -1)

    bt = 512
    grid = batch // bt
    full = lambda a: pl.BlockSpec(a.shape, lambda i: (0,) * a.ndim)

    partials = pl.pallas_call(
        functools.partial(_rqvae_tile, num_levels=num_levels, k=k),
        grid=(grid,),
        in_specs=[
            pl.BlockSpec((bt, in_dim), lambda i: (i, 0)),
            full(w1), full(row(b1)), full(w2), full(row(b2)),
            full(w3), full(row(b3)),
            full(v1), full(row(c1)), full(v2), full(row(c2)),
            full(v3), full(row(c3)),
            full(aug), full(cbhl),
        ],
        out_specs=pl.BlockSpec((1, 1, 128), lambda i: (i, 0, 0)),
        out_shape=jax.ShapeDtypeStruct((grid, 1, 128), _F32),
    )(x, w1, row(b1), w2, row(b2), w3, row(b3),
      v1, row(c1), v2, row(c2), v3, row(c3), aug, cbhl)

    return jnp.sum(partials[:, 0, 0]) / batch


# speedup vs baseline: 3.0565x; 1.3219x over previous
import functools

import jax
import jax.numpy as jnp
from jax.experimental import pallas as pl

_BETA = 0.25
_F32 = jnp.float32
_BF16 = jnp.bfloat16
_KC = 1024


def _mm(a, b):
    return jax.lax.dot_general(a, b, (((1,), (0,)), ((), ())),
                               preferred_element_type=_F32)


def _mm_t(a, b):
    return jax.lax.dot_general(a, b, (((1,), (1,)), ((), ())),
                               preferred_element_type=_F32)


def _augment_level(cb_ref, aug_ref, cbhl_ref):
    cb = cb_ref[0]
    c2 = jnp.sum(cb * cb, axis=1, keepdims=True)
    c2_hi = c2.astype(_BF16).astype(_F32)
    aug_ref[0] = jnp.concatenate(
        [cb * -2.0, c2_hi, c2 - c2_hi], axis=1).astype(_BF16)
    hi = cb.astype(_BF16)
    cbhl_ref[0] = jnp.concatenate(
        [hi, (cb - hi.astype(_F32)).astype(_BF16)], axis=1)


def _rqvae_tile(x_ref, w1_ref, b1_ref, w2_ref, b2_ref, w3_ref, b3_ref,
                v1_ref, c1_ref, v2_ref, c2_ref, v3_ref, c3_ref,
                aug_ref, cbhl_ref, out_ref, *, num_levels, k):
    xt = x_ref[...]
    h = jnp.maximum(_mm(xt, w1_ref[...]) + b1_ref[...], 0.0)
    h = jnp.maximum(_mm(h, w2_ref[...]) + b2_ref[...], 0.0)
    res = _mm(h, w3_ref[...]) + b3_ref[...]

    bt, dim = res.shape
    nchunks = k // _KC
    rq = jnp.zeros((bt, 1), dtype=_F32)
    z_hat = jnp.zeros_like(res)
    for l in range(num_levels):
        aug_res = jnp.concatenate(
            [res, jnp.ones((bt, 2), _F32)], axis=1).astype(_BF16)

        def _chunk(i, carry, l=l, aug_res=aug_res):
            best, hl = carry
            augc = aug_ref[l, pl.ds(i * _KC, _KC), :]
            score = _mm_t(aug_res, augc)
            m = jnp.min(score, axis=1, keepdims=True)
            oh = (score == m).astype(_BF16)
            hl_i = _mm(oh, cbhl_ref[l, pl.ds(i * _KC, _KC), :])
            upd = m < best
            return (jnp.where(upd, m, best), jnp.where(upd, hl_i, hl))

        best0 = jnp.full((bt, 1), jnp.inf, _F32)
        hl0 = jnp.zeros((bt, 2 * dim), _F32)
        _, hl = jax.lax.fori_loop(0, nchunks, _chunk, (best0, hl0), unroll=2)
        cw = hl[:, :dim] + hl[:, dim:]
        diff = cw - res
        rq = rq + jnp.sum(diff * diff, axis=1, keepdims=True)
        z_hat = z_hat + cw
        res = -diff

    rq = rq * (1.0 + _BETA)
    d = jnp.maximum(_mm(z_hat, v1_ref[...]) + c1_ref[...], 0.0)
    d = jnp.maximum(_mm(d, v2_ref[...]) + c2_ref[...], 0.0)
    x_hat = _mm(d, v3_ref[...]) + c3_ref[...]
    e = x_hat - xt
    recon = jnp.sum(e * e, axis=1, keepdims=True)
    out_ref[...] = jnp.full((1, 1, 128), jnp.sum(recon + rq), dtype=_F32)


@jax.jit
def kernel(x, temperature, enc_params, dec_params, codebooks):
    del temperature
    batch, in_dim = x.shape
    cbs = jnp.stack(codebooks, axis=0)
    num_levels, k, dim = cbs.shape

    aug, cbhl = pl.pallas_call(
        _augment_level,
        grid=(num_levels,),
        in_specs=[pl.BlockSpec((1, k, dim), lambda l: (l, 0, 0))],
        out_specs=[pl.BlockSpec((1, k, dim + 2), lambda l: (l, 0, 0)),
                   pl.BlockSpec((1, k, 2 * dim), lambda l: (l, 0, 0))],
        out_shape=[jax.ShapeDtypeStruct((num_levels, k, dim + 2), _BF16),
                   jax.ShapeDtypeStruct((num_levels, k, 2 * dim), _BF16)],
    )(cbs)

    (w1, b1), (w2, b2), (w3, b3) = enc_params
    (v1, c1), (v2, c2), (v3, c3) = dec_params
    row = lambda b: b.reshape(1, -1)

    bt = 512
    grid = batch // bt
    full = lambda a: pl.BlockSpec(a.shape, lambda i: (0,) * a.ndim)

    partials = pl.pallas_call(
        functools.partial(_rqvae_tile, num_levels=num_levels, k=k),
        grid=(grid,),
        in_specs=[
            pl.BlockSpec((bt, in_dim), lambda i: (i, 0)),
            full(w1), full(row(b1)), full(w2), full(row(b2)),
            full(w3), full(row(b3)),
            full(v1), full(row(c1)), full(v2), full(row(c2)),
            full(v3), full(row(c3)),
            full(aug), full(cbhl),
        ],
        out_specs=pl.BlockSpec((1, 1, 128), lambda i: (i, 0, 0)),
        out_shape=jax.ShapeDtypeStruct((grid, 1, 128), _F32),
    )(x, w1, row(b1), w2, row(b2), w3, row(b3),
      v1, row(c1), v2, row(c2), v3, row(c3), aug, cbhl)

    return jnp.sum(partials[:, 0, 0]) / batch
